# Initial kernel scaffold; baseline (speedup 1.0000x reference)
import jax
import jax.numpy as jnp
from jax.experimental import pallas as pl


def kernel(user_ids, user_countries, user_names, E_id, E_cty, E_name, W1, b1, W2, b2, W3, b3):
    raise NotImplementedError("write your pallas kernel here")


# speedup vs baseline: 1.6620x; 1.6620x over previous
import functools

import jax
import jax.numpy as jnp
from jax import lax
from jax.experimental import pallas as pl
from jax.experimental.pallas import tpu as pltpu
from jax.experimental.pallas import tpu_sc as plsc

_NC = 2
_NS = 16
_NW = _NC * _NS
_ICH = 128

_EPS = 1e-5


def _sc_gather_body(ids_hbm, cty_hbm, name_hbm, eid_hbm, ecty_hbm, ename_hbm,
                    out_id, out_cty, out_name,
                    idx_id, idx_cty, idx_name,
                    rows_id, rows_cty, rows_name, sem):
    nchunks = idx_id.shape[0]
    bpw = nchunks * _ICH
    wid = lax.axis_index("s") * _NC + lax.axis_index("c")
    base = wid * bpw
    pltpu.sync_copy(ids_hbm.at[wid], idx_id)
    pltpu.sync_copy(cty_hbm.at[wid], idx_cty)
    pltpu.sync_copy(name_hbm.at[wid], idx_name)
    copies = []
    for j in range(nchunks):
        dst = pl.ds(j * _ICH, _ICH)
        copies.append(pltpu.async_copy(eid_hbm.at[idx_id.at[j]], rows_id.at[dst], sem))
        copies.append(pltpu.async_copy(ecty_hbm.at[idx_cty.at[j]], rows_cty.at[dst], sem))
        copies.append(pltpu.async_copy(ename_hbm.at[idx_name.at[j]], rows_name.at[dst], sem))
    for c in copies:
        c.wait()
    out = pl.ds(base, bpw)
    pltpu.sync_copy(rows_id, out_id.at[out])
    pltpu.sync_copy(rows_cty, out_cty.at[out])
    pltpu.sync_copy(rows_name, out_name.at[out])


def _sc_gather(ids3, cty3, name3, E_id, E_cty, E_name, B, D):
    nchunks = B // (_NW * _ICH)
    bpw = nchunks * _ICH
    mesh = plsc.VectorSubcoreMesh(core_axis_name="c", subcore_axis_name="s")
    out_sd = jax.ShapeDtypeStruct((B, D), jnp.float32)
    f = pl.kernel(
        _sc_gather_body,
        out_type=(out_sd, out_sd, out_sd),
        mesh=mesh,
        scratch_types=[
            pltpu.VMEM((nchunks, _ICH), jnp.int32),
            pltpu.VMEM((nchunks, _ICH), jnp.int32),
            pltpu.VMEM((nchunks, _ICH), jnp.int32),
            pltpu.VMEM((bpw, D), jnp.float32),
            pltpu.VMEM((bpw, D), jnp.float32),
            pltpu.VMEM((bpw, D), jnp.float32),
            pltpu.SemaphoreType.DMA,
        ],
    )
    return f(ids3, cty3, name3, E_id, E_cty, E_name)


def _ln(x):
    mu = jnp.mean(x, axis=-1, keepdims=True)
    var = jnp.mean((x - mu) * (x - mu), axis=-1, keepdims=True)
    return (x - mu) * lax.rsqrt(var + _EPS)


def _gelu(x):
    return x * 0.5 * (1.0 + lax.erf(x * 0.7071067811865476))


def _mlp_body(id_ref, cty_ref, name_ref, w1a_ref, w1b_ref, w1c_ref, b1_ref,
              w2_ref, b2_ref, w3_ref, b3_ref, out_ref):
    f32 = jnp.float32
    h = (jnp.dot(id_ref[...], w1a_ref[...], preferred_element_type=f32)
         + jnp.dot(cty_ref[...], w1b_ref[...], preferred_element_type=f32)
         + jnp.dot(name_ref[...], w1c_ref[...], preferred_element_type=f32)
         + b1_ref[...])
    h = _gelu(_ln(h))
    h = jnp.dot(h, w2_ref[...], preferred_element_type=f32) + b2_ref[...]
    h = _gelu(_ln(h))
    h = jnp.dot(h, w3_ref[...], preferred_element_type=f32) + b3_ref[...]
    out_ref[...] = _gelu(h)


def _mlp(id_emb, cty_emb, name_emb, W1, b1, W2, b2, W3, b3, block_b):
    B, D = id_emb.shape
    H1 = W1.shape[1]
    H2 = W2.shape[1]
    H3 = W3.shape[1]
    grid = (B // block_b,)
    bspec = lambda shape, imap: pl.BlockSpec(shape, imap)
    data = lambda: bspec((block_b, D), lambda i: (i, 0))
    full = lambda r, c: bspec((r, c), lambda i: (0, 0))
    return pl.pallas_call(
        _mlp_body,
        grid=grid,
        in_specs=[
            data(), data(), data(),
            full(D, H1), full(D, H1), full(D, H1), full(1, H1),
            full(H1, H2), full(1, H2),
            full(H2, H3), full(1, H3),
        ],
        out_specs=bspec((block_b, H3), lambda i: (i, 0)),
        out_shape=jax.ShapeDtypeStruct((B, H3), jnp.float32),
    )(id_emb, cty_emb, name_emb,
      W1[:D], W1[D:2 * D], W1[2 * D:], b1.reshape(1, H1),
      W2, b2.reshape(1, H2), W3, b3.reshape(1, H3))


def kernel(user_ids, user_countries, user_names, E_id, E_cty, E_name,
           W1, b1, W2, b2, W3, b3):
    B = user_ids.shape[0]
    D = E_id.shape[1]
    nchunks = B // (_NW * _ICH)
    id_emb = jnp.take(E_id, user_ids, axis=0)
    cty_emb = jnp.take(E_cty, user_countries, axis=0)
    name_emb = jnp.take(E_name, user_names, axis=0)
    return _mlp(id_emb, cty_emb, name_emb, W1, b1, W2, b2, W3, b3, block_b=2048)
